# baseline (speedup 1.0000x reference)
import functools

import jax
import jax.numpy as jnp
from jax import lax
from jax.experimental import pallas as pl
from jax.experimental.pallas import tpu as pltpu
from jax.experimental.pallas import tpu_sc as plsc

NC = 2
NS = 16
CHUNK = 80
NBUF = 3
WOFF = 8


def _sc_segment_sum(x, packed, w):
    n, d = x.shape
    nw = NC * NS
    epw = packed.shape[0] // nw
    nchunk = epw // CHUNK
    assert nchunk * CHUNK == epw and (nchunk - 1) % NBUF == 0
    row_blk = 80
    nrow_blk = n // row_blk
    assert nrow_blk * row_blk == n

    mesh = plsc.VectorSubcoreMesh(core_axis_name="c", subcore_axis_name="s")

    @functools.partial(
        pl.kernel,
        out_type=jax.ShapeDtypeStruct((NC, n, d), jnp.float32),
        mesh=mesh,
        scratch_types=[
            pltpu.VMEM_SHARED((n, d), jnp.float32),
            pltpu.VMEM((epw,), jnp.int32),
            pltpu.VMEM((NBUF, CHUNK), jnp.int32),
            pltpu.VMEM((NBUF, CHUNK), jnp.int32),
            pltpu.VMEM((NBUF, CHUNK, d), jnp.float32),
            [pltpu.SemaphoreType.DMA] * NBUF,
            [pltpu.SemaphoreType.DMA] * NBUF,
        ],
        compiler_params=pltpu.CompilerParams(needs_layout_passes=False),
    )
    def sc_kernel(x_hbm, packed_hbm, w_hbm, zeros_hbm, part_hbm,
                  agg_sh, packed_v, src_v, dst_v, rows_v,
                  gsem, ssem):
        c = lax.axis_index("c")
        s = lax.axis_index("s")
        wid = c * NS + s
        ebase = wid * epw

        @pl.loop(s, nrow_blk, step=NS)
        def zero_loop(b):
            r0 = pl.multiple_of(b * row_blk, 8)
            pltpu.sync_copy(zeros_hbm.at[pl.ds(r0, row_blk)],
                            agg_sh.at[pl.ds(r0, row_blk)])
        plsc.subcore_barrier()

        pltpu.sync_copy(packed_hbm.at[pl.ds(ebase, epw)], packed_v)

        def unpack_idx(g, b):
            for i in range(CHUNK // 16):
                sl = pl.ds(i * 16, 16)
                v = packed_v[pl.ds(g * CHUNK + i * 16, 16)]
                src_v[b, sl] = v & jnp.int32(0xFFFF)
                dst_v[b, sl] = lax.shift_right_logical(v, jnp.int32(16))

        def start_gather(g, b):
            pltpu.async_copy(x_hbm.at[src_v.at[b]], rows_v.at[b], gsem[b])

        def wait_gather(g, b):
            pltpu.make_async_copy(
                x_hbm.at[src_v.at[b]], rows_v.at[b], gsem[b]).wait()

        def wait_scatter(b):
            pass

        def scale_rows(g, b):
            pass

        def start_scatter(b):
            pltpu.async_copy(rows_v.at[b], agg_sh.at[dst_v.at[b]],
                             ssem[b], add=True)

        for b in range(NBUF - 1):
            unpack_idx(b, b)
            start_gather(b, b)

        @pl.loop(0, nchunk - 1, step=NBUF)
        def edge_loop(j):
            for k in range(NBUF):
                g = j + k
                kp = (k + NBUF - 1) % NBUF
                wait_gather(g, k)

                @pl.when(g + NBUF - 1 < nchunk)
                def _():
                    @pl.when(g >= 1)
                    def _():
                        wait_scatter(kp)
                    unpack_idx(g + NBUF - 1, kp)
                    start_gather(g + NBUF - 1, kp)
                scale_rows(g, k)

        wait_gather(nchunk - 1, (nchunk - 1) % NBUF)
        scale_rows(nchunk - 1, (nchunk - 1) % NBUF)
        for b in range(NBUF):
            wait_scatter(b)

        plsc.subcore_barrier()

        @pl.loop(s, nrow_blk, step=NS)
        def flush_loop(b):
            r0 = pl.multiple_of(b * row_blk, 8)
            pltpu.sync_copy(agg_sh.at[pl.ds(r0, row_blk)],
                            part_hbm.at[c, pl.ds(r0, row_blk)])

    zeros = jnp.zeros((n, d), dtype=jnp.float32)
    return sc_kernel(x, packed, w, zeros)


def _tc_matmul_relu(partials, W0, W1, W2):
    _, n, d = partials.shape
    block_rows = 1000

    def body(p_ref, w0_ref, w1_ref, w2_ref, o_ref):
        pblk = p_ref[0] + p_ref[1]
        w = w0_ref[...] + w1_ref[...] + w2_ref[...]
        acc = jnp.dot(pblk, w, preferred_element_type=jnp.float32)
        o_ref[...] = jnp.maximum(acc, 0.0)

    return pl.pallas_call(
        body,
        grid=(n // block_rows,),
        in_specs=[
            pl.BlockSpec((2, block_rows, d), lambda i: (0, i, 0)),
            pl.BlockSpec((d, d), lambda i: (0, 0)),
            pl.BlockSpec((d, d), lambda i: (0, 0)),
            pl.BlockSpec((d, d), lambda i: (0, 0)),
        ],
        out_specs=pl.BlockSpec((block_rows, d), lambda i: (i, 0)),
        out_shape=jax.ShapeDtypeStruct((n, d), jnp.float32),
    )(partials, W0, W1, W2)


def kernel(x, edge_index, edge_weight, num_nodes, W0, W1, W2):
    e = edge_index.shape[1]
    nw = NC * NS
    epw = e // nw
    nchunk = (epw + CHUNK - 1) // CHUNK
    while (nchunk - 1) % NBUF:
        nchunk += 1
    epw_pad = nchunk * CHUNK
    pad = epw_pad - epw
    packed = jnp.bitwise_or(jnp.left_shift(edge_index[1], 16), edge_index[0])
    packed = jnp.pad(packed.reshape(nw, epw), ((0, 0), (0, pad))).reshape(-1)
    w_pad = jnp.pad(edge_weight.reshape(nw, epw), ((0, 0), (0, pad)))
    partials = _sc_segment_sum(x, packed, w_pad.reshape(-1))
    return _tc_matmul_relu(partials, W0, W1, W2)


# speedup vs baseline: 1.0505x; 1.0505x over previous
import functools

import jax
import jax.numpy as jnp
from jax import lax
from jax.experimental import pallas as pl
from jax.experimental.pallas import tpu as pltpu
from jax.experimental.pallas import tpu_sc as plsc

NC = 2
NS = 16
CHUNK = 80
NBUF = 2
WOFF = 8


def _sc_segment_sum(x, packed, w):
    n, d = x.shape
    nw = NC * NS
    epw = packed.shape[0] // nw
    nchunk = epw // CHUNK
    assert nchunk * CHUNK == epw and (nchunk - 1) % NBUF == 0
    row_blk = CHUNK
    nrow_blk = n // row_blk
    assert nrow_blk * row_blk == n

    mesh = plsc.VectorSubcoreMesh(core_axis_name="c", subcore_axis_name="s")

    @functools.partial(
        pl.kernel,
        out_type=jax.ShapeDtypeStruct((NC, n, d), jnp.float32),
        mesh=mesh,
        scratch_types=[
            pltpu.VMEM_SHARED((n, d), jnp.float32),
            pltpu.VMEM((epw,), jnp.int32),
            pltpu.VMEM((NBUF, CHUNK), jnp.int32),
            pltpu.VMEM((NBUF, CHUNK), jnp.int32),
            pltpu.VMEM((epw + WOFF,), jnp.float32),
            pltpu.VMEM((NBUF, CHUNK, d), jnp.float32),
            [pltpu.SemaphoreType.DMA] * NBUF,
            [pltpu.SemaphoreType.DMA] * NBUF,
        ],
        compiler_params=pltpu.CompilerParams(needs_layout_passes=False),
    )
    def sc_kernel(x_hbm, packed_hbm, w_hbm, part_hbm,
                  agg_sh, packed_v, src_v, dst_v, w_v, rows_v,
                  gsem, ssem):
        c = lax.axis_index("c")
        s = lax.axis_index("s")
        wid = c * NS + s
        ebase = wid * epw

        zvec = jnp.zeros((16,), dtype=jnp.float32)

        @pl.loop(0, CHUNK)
        def zbuf_loop(e):
            for r in range(d // 16):
                rows_v[0, e, pl.ds(r * 16, 16)] = zvec

        @pl.loop(s, nrow_blk, step=NS)
        def zero_loop(b):
            r0 = pl.multiple_of(b * row_blk, 8)
            pltpu.sync_copy(rows_v.at[0], agg_sh.at[pl.ds(r0, row_blk)])
        plsc.subcore_barrier()

        pltpu.sync_copy(packed_hbm.at[pl.ds(ebase, epw)], packed_v)
        pltpu.sync_copy(w_hbm.at[pl.ds(ebase, epw)],
                        w_v.at[pl.ds(WOFF, epw)])

        def unpack_idx(g, b):
            for i in range(CHUNK // 16):
                sl = pl.ds(i * 16, 16)
                v = packed_v[pl.ds(g * CHUNK + i * 16, 16)]
                src_v[b, sl] = v & jnp.int32(0xFFFF)
                dst_v[b, sl] = lax.shift_right_logical(v, jnp.int32(16))

        def start_gather(g, b):
            pltpu.async_copy(x_hbm.at[src_v.at[b]], rows_v.at[b], gsem[b])

        def wait_gather(g, b):
            pltpu.make_async_copy(
                x_hbm.at[src_v.at[b]], rows_v.at[b], gsem[b]).wait()

        def wait_scatter(b):
            pltpu.make_async_copy(
                rows_v.at[b], agg_sh.at[dst_v.at[b]], ssem[b]).wait()

        def scale_rows(g, b):
            wbase = g * CHUNK + WOFF
            for ei in range(CHUNK):
                wb = plsc.load_gather(
                    w_v, [jnp.full((16,), ei, dtype=jnp.int32) + wbase])
                for r in range(d // 16):
                    sl = pl.ds(r * 16, 16)
                    rows_v[b, ei, sl] = rows_v[b, ei, sl] * wb

        def start_scatter(b):
            pltpu.async_copy(rows_v.at[b], agg_sh.at[dst_v.at[b]],
                             ssem[b], add=True)

        for b in range(NBUF - 1):
            unpack_idx(b, b)
            start_gather(b, b)

        @pl.loop(0, nchunk - 1, step=NBUF)
        def edge_loop(j):
            for k in range(NBUF):
                g = j + k
                kp = (k + NBUF - 1) % NBUF

                @pl.when(g + NBUF - 1 < nchunk)
                def _():
                    @pl.when(g >= 1)
                    def _():
                        wait_scatter(kp)
                    unpack_idx(g + NBUF - 1, kp)
                    start_gather(g + NBUF - 1, kp)
                wait_gather(g, k)
                scale_rows(g, k)
                start_scatter(k)

        wait_gather(nchunk - 1, (nchunk - 1) % NBUF)
        scale_rows(nchunk - 1, (nchunk - 1) % NBUF)
        start_scatter((nchunk - 1) % NBUF)
        for b in range(NBUF):
            wait_scatter(b)

        plsc.subcore_barrier()

        @pl.loop(s, nrow_blk, step=NS)
        def flush_loop(b):
            r0 = pl.multiple_of(b * row_blk, 8)
            pltpu.sync_copy(agg_sh.at[pl.ds(r0, row_blk)],
                            part_hbm.at[c, pl.ds(r0, row_blk)])

    return sc_kernel(x, packed, w)


def _tc_matmul_relu(partials, W0, W1, W2):
    _, n, d = partials.shape
    block_rows = 1000

    def body(p_ref, w0_ref, w1_ref, w2_ref, o_ref):
        pblk = p_ref[0] + p_ref[1]
        w = w0_ref[...] + w1_ref[...] + w2_ref[...]
        acc = jnp.dot(pblk, w, preferred_element_type=jnp.float32)
        o_ref[...] = jnp.maximum(acc, 0.0)

    return pl.pallas_call(
        body,
        grid=(n // block_rows,),
        in_specs=[
            pl.BlockSpec((2, block_rows, d), lambda i: (0, i, 0)),
            pl.BlockSpec((d, d), lambda i: (0, 0)),
            pl.BlockSpec((d, d), lambda i: (0, 0)),
            pl.BlockSpec((d, d), lambda i: (0, 0)),
        ],
        out_specs=pl.BlockSpec((block_rows, d), lambda i: (i, 0)),
        out_shape=jax.ShapeDtypeStruct((n, d), jnp.float32),
    )(partials, W0, W1, W2)


def kernel(x, edge_index, edge_weight, num_nodes, W0, W1, W2):
    e = edge_index.shape[1]
    nw = NC * NS
    epw = e // nw
    nchunk = (epw + CHUNK - 1) // CHUNK
    while (nchunk - 1) % NBUF:
        nchunk += 1
    epw_pad = nchunk * CHUNK
    pad = epw_pad - epw
    packed = jnp.bitwise_or(jnp.left_shift(edge_index[1], 16), edge_index[0])
    packed = jnp.pad(packed.reshape(nw, epw), ((0, 0), (0, pad))).reshape(-1)
    w_pad = jnp.pad(edge_weight.reshape(nw, epw), ((0, 0), (0, pad)))
    partials = _sc_segment_sum(x, packed, w_pad.reshape(-1))
    return _tc_matmul_relu(partials, W0, W1, W2)


# speedup vs baseline: 1.6567x; 1.5771x over previous
import functools

import jax
import jax.numpy as jnp
from jax import lax
from jax.experimental import pallas as pl
from jax.experimental.pallas import tpu as pltpu
from jax.experimental.pallas import tpu_sc as plsc

NC = 2
NS = 16
CHUNK = 80
NBUF = 2
WOFF = 16


def _sc_segment_sum(x, packed, w):
    n, d = x.shape
    nw = NC * NS
    epw = packed.shape[0] // nw
    nchunk = epw // CHUNK
    assert nchunk * CHUNK == epw and (nchunk - 1) % NBUF == 0
    row_blk = 80
    nrow_blk = n // row_blk
    assert nrow_blk * row_blk == n

    mesh = plsc.VectorSubcoreMesh(core_axis_name="c", subcore_axis_name="s")

    @functools.partial(
        pl.kernel,
        out_type=jax.ShapeDtypeStruct((NC, n, d), jnp.float32),
        mesh=mesh,
        scratch_types=[
            pltpu.VMEM_SHARED((n, d), jnp.float32),
            pltpu.VMEM((epw,), jnp.int32),
            pltpu.VMEM((NBUF, CHUNK), jnp.int32),
            pltpu.VMEM((NBUF, CHUNK), jnp.int32),
            pltpu.VMEM((epw + WOFF,), jnp.float32),
            pltpu.VMEM((NBUF, CHUNK, d), jnp.float32),
            [pltpu.SemaphoreType.DMA] * NBUF,
            [pltpu.SemaphoreType.DMA] * NBUF,
        ],
        compiler_params=pltpu.CompilerParams(needs_layout_passes=False),
    )
    def sc_kernel(x_hbm, packed_hbm, w_hbm, zeros_hbm, part_hbm,
                  agg_sh, packed_v, src_v, dst_v, w_v, rows_v,
                  gsem, ssem):
        c = lax.axis_index("c")
        s = lax.axis_index("s")
        wid = c * NS + s
        ebase = wid * epw

        @pl.loop(s, nrow_blk, step=NS)
        def zero_loop(b):
            r0 = pl.multiple_of(b * row_blk, 8)
            pltpu.sync_copy(zeros_hbm.at[pl.ds(r0, row_blk)],
                            agg_sh.at[pl.ds(r0, row_blk)])
        plsc.subcore_barrier()

        pltpu.sync_copy(packed_hbm.at[pl.ds(ebase, epw)], packed_v)
        pltpu.sync_copy(w_hbm.at[pl.ds(ebase, epw)],
                        w_v.at[pl.ds(WOFF, epw)])

        def unpack_idx(g, b):
            for i in range(CHUNK // 16):
                sl = pl.ds(i * 16, 16)
                v = packed_v[pl.ds(g * CHUNK + i * 16, 16)]
                src_v[b, sl] = v & jnp.int32(0xFFFF)
                dst_v[b, sl] = lax.shift_right_logical(v, jnp.int32(16))

        def start_gather(g, b):
            pltpu.async_copy(x_hbm.at[src_v.at[b]], rows_v.at[b], gsem[b])

        def wait_gather(g, b):
            pltpu.make_async_copy(
                x_hbm.at[src_v.at[b]], rows_v.at[b], gsem[b]).wait()

        def wait_scatter(b):
            pltpu.make_async_copy(
                rows_v.at[b], agg_sh.at[dst_v.at[b]], ssem[b]).wait()

        bcast_dn = lax.GatherDimensionNumbers(
            offset_dims=(), collapsed_slice_dims=(0,), start_index_map=(0,))

        def scale_rows(g, b):
            wbase = g * CHUNK + WOFF
            for j16 in range(CHUNK // 16):
                w16 = w_v[pl.ds(wbase + j16 * 16, 16)]
                for i in range(16):
                    ei = j16 * 16 + i
                    wb = lax.gather(
                        w16, jnp.full((16, 1), i, jnp.int32), bcast_dn, (1,),
                        mode=lax.GatherScatterMode.PROMISE_IN_BOUNDS)
                    for r in range(d // 16):
                        sl = pl.ds(r * 16, 16)
                        rows_v[b, ei, sl] = rows_v[b, ei, sl] * wb

        def start_scatter(b):
            pltpu.async_copy(rows_v.at[b], agg_sh.at[dst_v.at[b]],
                             ssem[b], add=True)

        for b in range(NBUF - 1):
            unpack_idx(b, b)
            start_gather(b, b)

        @pl.loop(0, nchunk - 1, step=NBUF)
        def edge_loop(j):
            for k in range(NBUF):
                g = j + k
                kp = (k + NBUF - 1) % NBUF
                wait_gather(g, k)

                @pl.when(g + NBUF - 1 < nchunk)
                def _():
                    @pl.when(g >= 1)
                    def _():
                        wait_scatter(kp)
                    unpack_idx(g + NBUF - 1, kp)
                    start_gather(g + NBUF - 1, kp)
                scale_rows(g, k)
                start_scatter(k)

        wait_gather(nchunk - 1, (nchunk - 1) % NBUF)
        scale_rows(nchunk - 1, (nchunk - 1) % NBUF)
        start_scatter((nchunk - 1) % NBUF)
        for b in range(NBUF):
            wait_scatter(b)

        plsc.subcore_barrier()

        @pl.loop(s, nrow_blk, step=NS)
        def flush_loop(b):
            r0 = pl.multiple_of(b * row_blk, 8)
            pltpu.sync_copy(agg_sh.at[pl.ds(r0, row_blk)],
                            part_hbm.at[c, pl.ds(r0, row_blk)])

    zeros = jnp.zeros((n, d), dtype=jnp.float32)
    return sc_kernel(x, packed, w, zeros)


def _tc_matmul_relu(partials, W0, W1, W2):
    _, n, d = partials.shape
    block_rows = 1000

    def body(p_ref, w0_ref, w1_ref, w2_ref, o_ref):
        pblk = p_ref[0] + p_ref[1]
        w = w0_ref[...] + w1_ref[...] + w2_ref[...]
        acc = jnp.dot(pblk, w, preferred_element_type=jnp.float32)
        o_ref[...] = jnp.maximum(acc, 0.0)

    return pl.pallas_call(
        body,
        grid=(n // block_rows,),
        in_specs=[
            pl.BlockSpec((2, block_rows, d), lambda i: (0, i, 0)),
            pl.BlockSpec((d, d), lambda i: (0, 0)),
            pl.BlockSpec((d, d), lambda i: (0, 0)),
            pl.BlockSpec((d, d), lambda i: (0, 0)),
        ],
        out_specs=pl.BlockSpec((block_rows, d), lambda i: (i, 0)),
        out_shape=jax.ShapeDtypeStruct((n, d), jnp.float32),
    )(partials, W0, W1, W2)


def kernel(x, edge_index, edge_weight, num_nodes, W0, W1, W2):
    e = edge_index.shape[1]
    nw = NC * NS
    epw = e // nw
    nchunk = (epw + CHUNK - 1) // CHUNK
    while (nchunk - 1) % NBUF:
        nchunk += 1
    epw_pad = nchunk * CHUNK
    pad = epw_pad - epw
    packed = jnp.bitwise_or(jnp.left_shift(edge_index[1], 16), edge_index[0])
    packed = jnp.pad(packed.reshape(nw, epw), ((0, 0), (0, pad))).reshape(-1)
    w_pad = jnp.pad(edge_weight.reshape(nw, epw), ((0, 0), (0, pad)))
    partials = _sc_segment_sum(x, packed, w_pad.reshape(-1))
    return _tc_matmul_relu(partials, W0, W1, W2)


# speedup vs baseline: 1.7574x; 1.0608x over previous
import functools

import jax
import jax.numpy as jnp
from jax import lax
from jax.experimental import pallas as pl
from jax.experimental.pallas import tpu as pltpu
from jax.experimental.pallas import tpu_sc as plsc

NC = 2
NS = 16
CHUNK = 80
NBUF = 2
WOFF = 16


def _sc_segment_sum(x, packed, w):
    n, d = x.shape
    nw = NC * NS
    epw = packed.shape[0] // nw
    nchunk = epw // CHUNK
    assert nchunk * CHUNK == epw and (nchunk - 1) % NBUF == 0
    row_blk = 80
    nrow_blk = n // row_blk
    assert nrow_blk * row_blk == n

    mesh = plsc.VectorSubcoreMesh(core_axis_name="c", subcore_axis_name="s")

    @functools.partial(
        pl.kernel,
        out_type=jax.ShapeDtypeStruct((NC, n, d), jnp.float32),
        mesh=mesh,
        scratch_types=[
            pltpu.VMEM_SHARED((n, d), jnp.float32),
            pltpu.VMEM((epw,), jnp.int32),
            pltpu.VMEM((NBUF, CHUNK), jnp.int32),
            pltpu.VMEM((NBUF, CHUNK), jnp.int32),
            pltpu.VMEM((epw + WOFF,), jnp.float32),
            pltpu.VMEM((NBUF, CHUNK, d), jnp.float32),
            [pltpu.SemaphoreType.DMA] * NBUF,
            [pltpu.SemaphoreType.DMA] * NBUF,
            [pltpu.SemaphoreType.DMA] * NBUF,
        ],
        compiler_params=pltpu.CompilerParams(needs_layout_passes=False),
    )
    def sc_kernel(x_hbm, packed_hbm, w_hbm, zeros_hbm, part_hbm,
                  agg_sh, packed_v, src_v, dst_v, w_v, rows_v,
                  gsem, gsem2, ssem):
        c = lax.axis_index("c")
        s = lax.axis_index("s")
        wid = c * NS + s
        ebase = wid * epw

        @pl.loop(s, nrow_blk, step=NS)
        def zero_loop(b):
            r0 = pl.multiple_of(b * row_blk, 8)
            pltpu.sync_copy(zeros_hbm.at[pl.ds(r0, row_blk)],
                            agg_sh.at[pl.ds(r0, row_blk)])
        plsc.subcore_barrier()

        pltpu.sync_copy(packed_hbm.at[pl.ds(ebase, epw)], packed_v)
        pltpu.sync_copy(w_hbm.at[pl.ds(ebase, epw)],
                        w_v.at[pl.ds(WOFF, epw)])

        def unpack_idx(g, b):
            for i in range(CHUNK // 16):
                sl = pl.ds(i * 16, 16)
                v = packed_v[pl.ds(g * CHUNK + i * 16, 16)]
                src_v[b, sl] = v & jnp.int32(0xFFFF)
                dst_v[b, sl] = lax.shift_right_logical(v, jnp.int32(16))

        H = CHUNK // 2

        def start_gather(g, b):
            pltpu.async_copy(x_hbm.at[src_v.at[b, pl.ds(0, H)]],
                             rows_v.at[b, pl.ds(0, H)], gsem[b])
            pltpu.async_copy(x_hbm.at[src_v.at[b, pl.ds(H, H)]],
                             rows_v.at[b, pl.ds(H, H)], gsem2[b])

        def wait_gather(g, b):
            pltpu.make_async_copy(
                x_hbm.at[src_v.at[b, pl.ds(0, H)]],
                rows_v.at[b, pl.ds(0, H)], gsem[b]).wait()
            pltpu.make_async_copy(
                x_hbm.at[src_v.at[b, pl.ds(H, H)]],
                rows_v.at[b, pl.ds(H, H)], gsem2[b]).wait()

        def wait_scatter(b):
            pltpu.make_async_copy(
                rows_v.at[b], agg_sh.at[dst_v.at[b]], ssem[b]).wait()

        bcast_dn = lax.GatherDimensionNumbers(
            offset_dims=(), collapsed_slice_dims=(0,), start_index_map=(0,))

        def scale_rows(g, b):
            wbase = g * CHUNK + WOFF
            for j16 in range(CHUNK // 16):
                w16 = w_v[pl.ds(wbase + j16 * 16, 16)]
                for i in range(16):
                    ei = j16 * 16 + i
                    wb = lax.gather(
                        w16, jnp.full((16, 1), i, jnp.int32), bcast_dn, (1,),
                        mode=lax.GatherScatterMode.PROMISE_IN_BOUNDS)
                    for r in range(d // 16):
                        sl = pl.ds(r * 16, 16)
                        rows_v[b, ei, sl] = rows_v[b, ei, sl] * wb

        def start_scatter(b):
            pltpu.async_copy(rows_v.at[b], agg_sh.at[dst_v.at[b]],
                             ssem[b], add=True)

        for b in range(NBUF - 1):
            unpack_idx(b, b)
            start_gather(b, b)

        @pl.loop(0, nchunk - 1, step=NBUF)
        def edge_loop(j):
            for k in range(NBUF):
                g = j + k
                kp = (k + NBUF - 1) % NBUF
                wait_gather(g, k)

                @pl.when(g + NBUF - 1 < nchunk)
                def _():
                    @pl.when(g >= 1)
                    def _():
                        wait_scatter(kp)
                    unpack_idx(g + NBUF - 1, kp)
                    start_gather(g + NBUF - 1, kp)
                scale_rows(g, k)
                start_scatter(k)

        wait_gather(nchunk - 1, (nchunk - 1) % NBUF)
        scale_rows(nchunk - 1, (nchunk - 1) % NBUF)
        start_scatter((nchunk - 1) % NBUF)
        for b in range(NBUF):
            wait_scatter(b)

        plsc.subcore_barrier()

        @pl.loop(s, nrow_blk, step=NS)
        def flush_loop(b):
            r0 = pl.multiple_of(b * row_blk, 8)
            pltpu.sync_copy(agg_sh.at[pl.ds(r0, row_blk)],
                            part_hbm.at[c, pl.ds(r0, row_blk)])

    zeros = jnp.zeros((n, d), dtype=jnp.float32)
    return sc_kernel(x, packed, w, zeros)


def _tc_matmul_relu(partials, W0, W1, W2):
    _, n, d = partials.shape
    block_rows = 1000

    def body(p_ref, w0_ref, w1_ref, w2_ref, o_ref):
        pblk = p_ref[0] + p_ref[1]
        w = w0_ref[...] + w1_ref[...] + w2_ref[...]
        acc = jnp.dot(pblk, w, preferred_element_type=jnp.float32)
        o_ref[...] = jnp.maximum(acc, 0.0)

    return pl.pallas_call(
        body,
        grid=(n // block_rows,),
        in_specs=[
            pl.BlockSpec((2, block_rows, d), lambda i: (0, i, 0)),
            pl.BlockSpec((d, d), lambda i: (0, 0)),
            pl.BlockSpec((d, d), lambda i: (0, 0)),
            pl.BlockSpec((d, d), lambda i: (0, 0)),
        ],
        out_specs=pl.BlockSpec((block_rows, d), lambda i: (i, 0)),
        out_shape=jax.ShapeDtypeStruct((n, d), jnp.float32),
    )(partials, W0, W1, W2)


def kernel(x, edge_index, edge_weight, num_nodes, W0, W1, W2):
    e = edge_index.shape[1]
    nw = NC * NS
    epw = e // nw
    nchunk = (epw + CHUNK - 1) // CHUNK
    while (nchunk - 1) % NBUF:
        nchunk += 1
    epw_pad = nchunk * CHUNK
    pad = epw_pad - epw
    packed = jnp.bitwise_or(jnp.left_shift(edge_index[1], 16), edge_index[0])
    packed = jnp.pad(packed.reshape(nw, epw), ((0, 0), (0, pad))).reshape(-1)
    w_pad = jnp.pad(edge_weight.reshape(nw, epw), ((0, 0), (0, pad)))
    partials = _sc_segment_sum(x, packed, w_pad.reshape(-1))
    return _tc_matmul_relu(partials, W0, W1, W2)


# speedup vs baseline: 1.7629x; 1.0031x over previous
import functools

import jax
import jax.numpy as jnp
from jax import lax
from jax.experimental import pallas as pl
from jax.experimental.pallas import tpu as pltpu
from jax.experimental.pallas import tpu_sc as plsc

NC = 2
NS = 16
CHUNK = 80
NBUF = 2
WOFF = 16


def _sc_segment_sum(x, packed, w):
    n, d = x.shape
    nw = NC * NS
    epw = packed.shape[0] // nw
    nchunk = epw // CHUNK
    assert nchunk * CHUNK == epw and (nchunk - 1) % NBUF == 0
    row_blk = 80
    nrow_blk = n // row_blk
    assert nrow_blk * row_blk == n

    mesh = plsc.VectorSubcoreMesh(core_axis_name="c", subcore_axis_name="s")

    @functools.partial(
        pl.kernel,
        out_type=jax.ShapeDtypeStruct((NC, n, d), jnp.float32),
        mesh=mesh,
        scratch_types=[
            pltpu.VMEM_SHARED((n, d), jnp.float32),
            pltpu.VMEM((epw,), jnp.int32),
            pltpu.VMEM((NBUF, CHUNK), jnp.int32),
            pltpu.VMEM((NBUF, CHUNK), jnp.int32),
            pltpu.VMEM((epw + WOFF,), jnp.float32),
            pltpu.VMEM((NBUF, CHUNK, d), jnp.float32),
            [pltpu.SemaphoreType.DMA] * NBUF,
            [pltpu.SemaphoreType.DMA] * NBUF,
            [pltpu.SemaphoreType.DMA] * NBUF,
        ],
        compiler_params=pltpu.CompilerParams(needs_layout_passes=False),
    )
    def sc_kernel(x_hbm, packed_hbm, w_hbm, zeros_hbm, part_hbm,
                  agg_sh, packed_v, src_v, dst_v, w_v, rows_v,
                  gsem, gsem2, ssem):
        c = lax.axis_index("c")
        s = lax.axis_index("s")
        wid = c * NS + s
        ebase = wid * epw

        @pl.loop(s, nrow_blk, step=NS)
        def zero_loop(b):
            r0 = pl.multiple_of(b * row_blk, 8)
            pltpu.sync_copy(zeros_hbm.at[pl.ds(r0, row_blk)],
                            agg_sh.at[pl.ds(r0, row_blk)])
        plsc.subcore_barrier()

        pltpu.sync_copy(packed_hbm.at[pl.ds(ebase, epw)], packed_v)
        pltpu.sync_copy(w_hbm.at[pl.ds(ebase, epw)],
                        w_v.at[pl.ds(WOFF, epw)])

        def unpack_idx(g, b):
            for i in range(CHUNK // 16):
                sl = pl.ds(i * 16, 16)
                v = packed_v[pl.ds(g * CHUNK + i * 16, 16)]
                src_v[b, sl] = v & jnp.int32(0xFFFF)
                dst_v[b, sl] = lax.shift_right_logical(v, jnp.int32(16))

        H = CHUNK // 4

        def start_gather(g, b):
            for q in range(2):
                pltpu.async_copy(
                    x_hbm.at[src_v.at[b, pl.ds(q * 2 * H, H)]],
                    rows_v.at[b, pl.ds(q * 2 * H, H)], gsem[b])
                pltpu.async_copy(
                    x_hbm.at[src_v.at[b, pl.ds((q * 2 + 1) * H, H)]],
                    rows_v.at[b, pl.ds((q * 2 + 1) * H, H)], gsem2[b])

        def wait_gather(g, b):
            for q in range(2):
                pltpu.make_async_copy(
                    x_hbm.at[src_v.at[b, pl.ds(q * 2 * H, H)]],
                    rows_v.at[b, pl.ds(q * 2 * H, H)], gsem[b]).wait()
                pltpu.make_async_copy(
                    x_hbm.at[src_v.at[b, pl.ds((q * 2 + 1) * H, H)]],
                    rows_v.at[b, pl.ds((q * 2 + 1) * H, H)], gsem2[b]).wait()

        def wait_scatter(b):
            pltpu.make_async_copy(
                rows_v.at[b], agg_sh.at[dst_v.at[b]], ssem[b]).wait()

        bcast_dn = lax.GatherDimensionNumbers(
            offset_dims=(), collapsed_slice_dims=(0,), start_index_map=(0,))

        def scale_rows(g, b):
            wbase = g * CHUNK + WOFF
            for j16 in range(CHUNK // 16):
                w16 = w_v[pl.ds(wbase + j16 * 16, 16)]
                for i in range(16):
                    ei = j16 * 16 + i
                    wb = lax.gather(
                        w16, jnp.full((16, 1), i, jnp.int32), bcast_dn, (1,),
                        mode=lax.GatherScatterMode.PROMISE_IN_BOUNDS)
                    for r in range(d // 16):
                        sl = pl.ds(r * 16, 16)
                        rows_v[b, ei, sl] = rows_v[b, ei, sl] * wb

        def start_scatter(b):
            pltpu.async_copy(rows_v.at[b], agg_sh.at[dst_v.at[b]],
                             ssem[b], add=True)

        for b in range(NBUF - 1):
            unpack_idx(b, b)
            start_gather(b, b)

        @pl.loop(0, nchunk - 1, step=NBUF)
        def edge_loop(j):
            for k in range(NBUF):
                g = j + k
                kp = (k + NBUF - 1) % NBUF
                wait_gather(g, k)

                @pl.when(g + NBUF - 1 < nchunk)
                def _():
                    @pl.when(g >= 1)
                    def _():
                        wait_scatter(kp)
                    unpack_idx(g + NBUF - 1, kp)
                    start_gather(g + NBUF - 1, kp)
                scale_rows(g, k)
                start_scatter(k)

        wait_gather(nchunk - 1, (nchunk - 1) % NBUF)
        scale_rows(nchunk - 1, (nchunk - 1) % NBUF)
        start_scatter((nchunk - 1) % NBUF)
        for b in range(NBUF):
            wait_scatter(b)

        plsc.subcore_barrier()

        @pl.loop(s, nrow_blk, step=NS)
        def flush_loop(b):
            r0 = pl.multiple_of(b * row_blk, 8)
            pltpu.sync_copy(agg_sh.at[pl.ds(r0, row_blk)],
                            part_hbm.at[c, pl.ds(r0, row_blk)])

    zeros = jnp.zeros((n, d), dtype=jnp.float32)
    return sc_kernel(x, packed, w, zeros)


def _tc_matmul_relu(partials, W0, W1, W2):
    _, n, d = partials.shape
    block_rows = 1000

    def body(p_ref, w0_ref, w1_ref, w2_ref, o_ref):
        pblk = p_ref[0] + p_ref[1]
        w = w0_ref[...] + w1_ref[...] + w2_ref[...]
        acc = jnp.dot(pblk, w, preferred_element_type=jnp.float32)
        o_ref[...] = jnp.maximum(acc, 0.0)

    return pl.pallas_call(
        body,
        grid=(n // block_rows,),
        in_specs=[
            pl.BlockSpec((2, block_rows, d), lambda i: (0, i, 0)),
            pl.BlockSpec((d, d), lambda i: (0, 0)),
            pl.BlockSpec((d, d), lambda i: (0, 0)),
            pl.BlockSpec((d, d), lambda i: (0, 0)),
        ],
        out_specs=pl.BlockSpec((block_rows, d), lambda i: (i, 0)),
        out_shape=jax.ShapeDtypeStruct((n, d), jnp.float32),
    )(partials, W0, W1, W2)


def kernel(x, edge_index, edge_weight, num_nodes, W0, W1, W2):
    e = edge_index.shape[1]
    nw = NC * NS
    epw = e // nw
    nchunk = (epw + CHUNK - 1) // CHUNK
    while (nchunk - 1) % NBUF:
        nchunk += 1
    epw_pad = nchunk * CHUNK
    pad = epw_pad - epw
    packed = jnp.bitwise_or(jnp.left_shift(edge_index[1], 16), edge_index[0])
    packed = jnp.pad(packed.reshape(nw, epw), ((0, 0), (0, pad))).reshape(-1)
    w_pad = jnp.pad(edge_weight.reshape(nw, epw), ((0, 0), (0, pad)))
    partials = _sc_segment_sum(x, packed, w_pad.reshape(-1))
    return _tc_matmul_relu(partials, W0, W1, W2)
